# Initial kernel scaffold; baseline (speedup 1.0000x reference)
import jax
import jax.numpy as jnp
from jax.experimental import pallas as pl


def kernel(lats, x_seps, y_seps):
    raise NotImplementedError("write your pallas kernel here")


# speedup vs baseline: 13.4002x; 13.4002x over previous
import jax
import jax.numpy as jnp
from jax import lax
from jax.experimental import pallas as pl

N = 8192
L = 64
P = L * L
B = 256


def _mask_body(y_ref, x_ref, out_ref):
    y = y_ref[...]
    x = x_ref[...]
    sep = y * L + x
    flat = lax.broadcasted_iota(jnp.int32, (B, P), 1)
    out_ref[...] = jnp.where(
        flat == sep, jnp.float32(-1.0),
        jnp.where(flat == 0, jnp.float32(1.0), jnp.float32(0.0)))


def kernel(lats, x_seps, y_seps):
    n = lats.shape[0]
    y2 = y_seps.astype(jnp.int32).reshape(n, 1)
    x2 = x_seps.astype(jnp.int32).reshape(n, 1)
    flat_out = pl.pallas_call(
        _mask_body,
        grid=(n // B,),
        in_specs=[
            pl.BlockSpec((B, 1), lambda i: (i, 0)),
            pl.BlockSpec((B, 1), lambda i: (i, 0)),
        ],
        out_specs=pl.BlockSpec((B, P), lambda i: (i, 0)),
        out_shape=jax.ShapeDtypeStruct((n, P), lats.dtype),
    )(y2, x2)
    return flat_out.reshape(n, 1, L, L)
